# baseline (speedup 1.0000x reference)
import functools

import jax
import jax.numpy as jnp
from jax import lax
from jax.experimental import pallas as pl
from jax.experimental.pallas import tpu as pltpu
from jax.experimental.pallas import tpu_sc as plsc

N = 10000
E = 320000
D = 128
NC, NS = 2, 16
NW = NC * NS
EPW = E // NW
CHUNK = 128
NCHUNK = 80
EPW_PAD = NCHUNK * CHUNK
NCH2 = NCHUNK // 2
NPAD = N + 8
LANES = 16
RPS = 624
RPS_LAST = N - (NS - 1) * RPS


def _sc_aggregate(xp, packed_r, zeros):
    mesh = plsc.VectorSubcoreMesh(core_axis_name="c", subcore_axis_name="s")

    @functools.partial(
        pl.kernel,
        out_type=jax.ShapeDtypeStruct((NC, N, D), jnp.float32),
        mesh=mesh,
        scratch_types=[
            pltpu.VMEM((NCHUNK, CHUNK), jnp.int32),
            pltpu.VMEM((CHUNK,), jnp.int32),
            pltpu.VMEM((CHUNK,), jnp.int32),
            pltpu.VMEM((CHUNK,), jnp.int32),
            pltpu.VMEM((CHUNK,), jnp.int32),
            pltpu.VMEM((CHUNK, D), jnp.float32),
            pltpu.VMEM((CHUNK, D), jnp.float32),
            pltpu.VMEM_SHARED((N, D), jnp.float32),
            pltpu.SemaphoreType.DMA,
            pltpu.SemaphoreType.DMA,
        ],
    )
    def k(x_hbm, pk_hbm, z_hbm, out_hbm, pk_v, s0_v, d0_v, s1_v, d1_v,
          rows0_v, rows1_v, agg_sh, sem0, sem1):
        c = lax.axis_index("c")
        s = lax.axis_index("s")
        wid = c * NS + s
        pltpu.sync_copy(pk_hbm.at[wid], pk_v)
        r0 = s * RPS

        @pl.when(s < NS - 1)
        def _():
            pltpu.sync_copy(z_hbm.at[pl.ds(0, RPS)], agg_sh.at[pl.ds(r0, RPS)])

        @pl.when(s == NS - 1)
        def _():
            pltpu.sync_copy(
                z_hbm.at[pl.ds(0, RPS_LAST)],
                agg_sh.at[pl.ds((NS - 1) * RPS, RPS_LAST)],
            )

        plsc.subcore_barrier()

        def unpack(j, sbuf, dbuf):
            for kk in range(CHUNK // LANES):
                pk = pk_v[j, pl.ds(kk * LANES, LANES)]
                sbuf[pl.ds(kk * LANES, LANES)] = lax.shift_right_logical(pk, 16)
                dbuf[pl.ds(kk * LANES, LANES)] = lax.bitwise_and(pk, 0xFFFF)

        def gather(sbuf, buf, sem):
            pltpu.async_copy(x_hbm.at[sbuf], buf, sem)

        def wait(buf, sem):
            pltpu.make_async_copy(x_hbm.at[pl.ds(0, CHUNK)], buf, sem).wait()

        last = NCHUNK - 1
        unpack(0, s0_v, d0_v)
        gather(s0_v, rows0_v, sem0)
        unpack(1, s1_v, d1_v)
        gather(s1_v, rows1_v, sem1)

        def body(i, carry):
            j0 = 2 * i
            wait(rows0_v, sem0)
            pltpu.sync_copy(rows0_v, agg_sh.at[d0_v], add=True)
            unpack(jnp.minimum(j0 + 2, last), s0_v, d0_v)
            gather(s0_v, rows0_v, sem0)
            wait(rows1_v, sem1)
            pltpu.sync_copy(rows1_v, agg_sh.at[d1_v], add=True)
            unpack(jnp.minimum(j0 + 3, last), s1_v, d1_v)
            gather(s1_v, rows1_v, sem1)
            return carry

        lax.fori_loop(0, NCH2, body, 0)
        wait(rows0_v, sem0)
        wait(rows1_v, sem1)
        plsc.subcore_barrier()

        @pl.when(s < NS - 1)
        def _():
            pltpu.sync_copy(
                agg_sh.at[pl.ds(r0, RPS)], out_hbm.at[c, pl.ds(r0, RPS)]
            )

        @pl.when(s == NS - 1)
        def _():
            pltpu.sync_copy(
                agg_sh.at[pl.ds((NS - 1) * RPS, RPS_LAST)],
                out_hbm.at[c, pl.ds((NS - 1) * RPS, RPS_LAST)],
            )

    return k(xp, packed_r, zeros)


def _tc_finish(x, parts, w2d, W1, b1, W2, b2):
    def body(x_ref, p_ref, w_ref, w1_ref, b1_ref, w2_ref, b2_ref, out_ref):
        h = x_ref[...] + p_ref[0] + p_ref[1]
        t = jnp.dot(h, w1_ref[...].T, preferred_element_type=jnp.float32)
        t = jnp.maximum(t + b1_ref[...], 0.0)
        wv = w_ref[...]
        v = jnp.sum(t * wv, axis=0, keepdims=True)
        sw = jnp.sum(wv)
        out = jnp.dot(v, w2_ref[...].T, preferred_element_type=jnp.float32)
        out_ref[...] = out + sw * b2_ref[...]

    return pl.pallas_call(
        body,
        out_shape=jax.ShapeDtypeStruct((1, D), jnp.float32),
    )(x, parts, w2d, W1, b1, W2, b2)


def kernel(x, edge_index, weights, W1, b1, W2, b2):
    src = edge_index[0].reshape(NW, EPW)
    dst = edge_index[1].reshape(NW, EPW)
    packed = jnp.left_shift(src, 16) | dst
    packed = jnp.pad(
        packed, ((0, 0), (0, EPW_PAD - EPW)), constant_values=N << 16
    ).reshape(NW, NCHUNK, CHUNK)
    xp = jnp.pad(x, ((0, NPAD - N), (0, 0)))
    zeros = jnp.zeros((RPS_LAST, D), jnp.float32)
    parts = _sc_aggregate(xp, packed, zeros)
    out = _tc_finish(x, parts, weights.reshape(N, 1), W1, b1, W2, b2)
    return out.reshape(1, 1, D)


# speedup vs baseline: 1.4768x; 1.4768x over previous
import functools

import jax
import jax.numpy as jnp
from jax import lax
from jax.experimental import pallas as pl
from jax.experimental.pallas import tpu as pltpu
from jax.experimental.pallas import tpu_sc as plsc

N = 10000
E = 320000
D = 128
NC, NS = 2, 16
NW = NC * NS
EPW = E // NW
CHUNK = 128
NCHUNK = 80
EPW_PAD = NCHUNK * CHUNK
NCH2 = NCHUNK // 2
NPAD = N + 8
LANES = 16
RPS = 624
RPS_LAST = N - (NS - 1) * RPS


def _sc_aggregate(xp, packed_r, zeros):
    mesh = plsc.VectorSubcoreMesh(core_axis_name="c", subcore_axis_name="s")

    @functools.partial(
        pl.kernel,
        out_type=jax.ShapeDtypeStruct((NC, N, D), jnp.float32),
        mesh=mesh,
        scratch_types=[
            pltpu.VMEM((NCHUNK, CHUNK), jnp.int32),
            pltpu.VMEM((CHUNK,), jnp.int32),
            pltpu.VMEM((CHUNK,), jnp.int32),
            pltpu.VMEM((CHUNK,), jnp.int32),
            pltpu.VMEM((CHUNK,), jnp.int32),
            pltpu.VMEM((CHUNK, D), jnp.float32),
            pltpu.VMEM((CHUNK, D), jnp.float32),
            pltpu.VMEM_SHARED((N, D), jnp.float32),
            pltpu.SemaphoreType.DMA,
            pltpu.SemaphoreType.DMA,
        ],
    )
    def k(x_hbm, pk_hbm, z_hbm, out_hbm, pk_v, s0_v, d0_v, s1_v, d1_v,
          rows0_v, rows1_v, agg_sh, sem0, sem1):
        c = lax.axis_index("c")
        s = lax.axis_index("s")
        wid = c * NS + s
        pltpu.sync_copy(pk_hbm.at[wid], pk_v)
        r0 = s * RPS

        @pl.when(s < NS - 1)
        def _():
            pltpu.sync_copy(z_hbm.at[pl.ds(0, RPS)], agg_sh.at[pl.ds(r0, RPS)])

        @pl.when(s == NS - 1)
        def _():
            pltpu.sync_copy(
                z_hbm.at[pl.ds(0, RPS_LAST)],
                agg_sh.at[pl.ds((NS - 1) * RPS, RPS_LAST)],
            )

        plsc.subcore_barrier()

        def unpack(j, sbuf, dbuf):
            for kk in range(CHUNK // LANES):
                pk = pk_v[j, pl.ds(kk * LANES, LANES)]
                sbuf[pl.ds(kk * LANES, LANES)] = lax.shift_right_logical(pk, 16)
                dbuf[pl.ds(kk * LANES, LANES)] = lax.bitwise_and(pk, 0xFFFF)

        def gather(sbuf, buf, sem):
            pltpu.async_copy(x_hbm.at[sbuf], buf, sem)

        def wait(buf, sem):
            pltpu.make_async_copy(x_hbm.at[pl.ds(0, CHUNK)], buf, sem).wait()

        last = NCHUNK - 1

        def body(j, carry):
            unpack(j, s0_v, d0_v)
            pltpu.async_copy(x_hbm.at[s0_v], rows0_v, sem0).wait()
            pltpu.sync_copy(rows0_v, agg_sh.at[d0_v], add=True)
            return carry

        lax.fori_loop(0, NCHUNK, body, 0)
        plsc.subcore_barrier()

        @pl.when(s < NS - 1)
        def _():
            pltpu.sync_copy(
                agg_sh.at[pl.ds(r0, RPS)], out_hbm.at[c, pl.ds(r0, RPS)]
            )

        @pl.when(s == NS - 1)
        def _():
            pltpu.sync_copy(
                agg_sh.at[pl.ds((NS - 1) * RPS, RPS_LAST)],
                out_hbm.at[c, pl.ds((NS - 1) * RPS, RPS_LAST)],
            )

    return k(xp, packed_r, zeros)


def _tc_finish(x, parts, w2d, W1, b1, W2, b2):
    def body(x_ref, p_ref, w_ref, w1_ref, b1_ref, w2_ref, b2_ref, out_ref):
        h = x_ref[...] + p_ref[0] + p_ref[1]
        t = jnp.dot(h, w1_ref[...].T, preferred_element_type=jnp.float32)
        t = jnp.maximum(t + b1_ref[...], 0.0)
        wv = w_ref[...]
        v = jnp.sum(t * wv, axis=0, keepdims=True)
        sw = jnp.sum(wv)
        out = jnp.dot(v, w2_ref[...].T, preferred_element_type=jnp.float32)
        out_ref[...] = out + sw * b2_ref[...]

    return pl.pallas_call(
        body,
        out_shape=jax.ShapeDtypeStruct((1, D), jnp.float32),
    )(x, parts, w2d, W1, b1, W2, b2)


def kernel(x, edge_index, weights, W1, b1, W2, b2):
    src = edge_index[0].reshape(NW, EPW)
    dst = edge_index[1].reshape(NW, EPW)
    packed = jnp.left_shift(src, 16) | dst
    packed = jnp.pad(
        packed, ((0, 0), (0, EPW_PAD - EPW)), constant_values=N << 16
    ).reshape(NW, NCHUNK, CHUNK)
    xp = jnp.pad(x, ((0, NPAD - N), (0, 0)))
    zeros = jnp.zeros((RPS_LAST, D), jnp.float32)
    parts = _sc_aggregate(xp, packed, zeros)
    out = _tc_finish(x, parts, weights.reshape(N, 1), W1, b1, W2, b2)
    return out.reshape(1, 1, D)


# speedup vs baseline: 3.0506x; 2.0656x over previous
import functools

import jax
import jax.numpy as jnp
from jax import lax
from jax.experimental import pallas as pl
from jax.experimental.pallas import tpu as pltpu
from jax.experimental.pallas import tpu_sc as plsc

N = 10000
E = 320000
D = 128
NC, NS = 2, 16
NW = NC * NS
EPW = E // NW
CHUNK = 80
NCHUNK = EPW // CHUNK
RPS = 624
RPS_LAST = N - (NS - 1) * RPS


def _sc_aggregate(x, edges_r, zeros):
    mesh = plsc.VectorSubcoreMesh(core_axis_name="c", subcore_axis_name="s")

    @functools.partial(
        pl.kernel,
        out_type=jax.ShapeDtypeStruct((NC, N, D), jnp.float32),
        mesh=mesh,
        compiler_params=pltpu.CompilerParams(use_tc_tiling_on_sc=False),
        scratch_types=[
            pltpu.VMEM((2, NCHUNK, CHUNK), jnp.int32),
            pltpu.VMEM((CHUNK, D), jnp.float32),
            pltpu.VMEM_SHARED((N, D), jnp.float32),
            pltpu.SemaphoreType.DMA,
        ],
    )
    def k(x_hbm, e_hbm, z_hbm, out_hbm, idx_v, rows_v, agg_sh, sem):
        c = lax.axis_index("c")
        s = lax.axis_index("s")
        wid = c * NS + s
        pltpu.sync_copy(e_hbm.at[wid], idx_v)
        r0 = s * RPS

        @pl.when(s < NS - 1)
        def _():
            pltpu.sync_copy(z_hbm.at[pl.ds(0, RPS)], agg_sh.at[pl.ds(r0, RPS)])

        @pl.when(s == NS - 1)
        def _():
            pltpu.sync_copy(
                z_hbm.at[pl.ds(0, RPS_LAST)],
                agg_sh.at[pl.ds((NS - 1) * RPS, RPS_LAST)],
            )

        plsc.subcore_barrier()

        def body(j, carry):
            pltpu.async_copy(x_hbm.at[idx_v.at[0, j]], rows_v, sem).wait()
            pltpu.sync_copy(rows_v, agg_sh.at[idx_v.at[1, j]], add=True)
            return carry

        lax.fori_loop(0, NCHUNK, body, 0)
        plsc.subcore_barrier()

        @pl.when(s < NS - 1)
        def _():
            pltpu.sync_copy(
                agg_sh.at[pl.ds(r0, RPS)], out_hbm.at[c, pl.ds(r0, RPS)]
            )

        @pl.when(s == NS - 1)
        def _():
            pltpu.sync_copy(
                agg_sh.at[pl.ds((NS - 1) * RPS, RPS_LAST)],
                out_hbm.at[c, pl.ds((NS - 1) * RPS, RPS_LAST)],
            )

    return k(x, edges_r, zeros)


def _tc_finish(x, parts, w2d, W1, b1, W2, b2):
    def body(x_ref, p_ref, w_ref, w1_ref, b1_ref, w2_ref, b2_ref, out_ref):
        h = x_ref[...] + p_ref[0] + p_ref[1]
        t = jnp.dot(h, w1_ref[...].T, preferred_element_type=jnp.float32)
        t = jnp.maximum(t + b1_ref[...], 0.0)
        wv = w_ref[...]
        v = jnp.sum(t * wv, axis=0, keepdims=True)
        sw = jnp.sum(wv)
        out = jnp.dot(v, w2_ref[...].T, preferred_element_type=jnp.float32)
        out_ref[...] = out + sw * b2_ref[...]

    return pl.pallas_call(
        body,
        out_shape=jax.ShapeDtypeStruct((1, D), jnp.float32),
    )(x, parts, w2d, W1, b1, W2, b2)


def kernel(x, edge_index, weights, W1, b1, W2, b2):
    edges_r = edge_index.reshape(2, NW, NCHUNK, CHUNK).transpose(1, 0, 2, 3)
    zeros = jnp.zeros((RPS_LAST, D), jnp.float32)
    parts = _sc_aggregate(x, edges_r, zeros)
    out = _tc_finish(x, parts, weights.reshape(N, 1), W1, b1, W2, b2)
    return out.reshape(1, 1, D)


# speedup vs baseline: 4.7098x; 1.5439x over previous
import functools

import jax
import jax.numpy as jnp
from jax import lax
from jax.experimental import pallas as pl
from jax.experimental.pallas import tpu as pltpu
from jax.experimental.pallas import tpu_sc as plsc

N = 10000
E = 320000
D = 128
NC, NS = 2, 16
NW = NC * NS
EPW = E // NW
CHUNK = 80
NCHUNK = EPW // CHUNK
RPS = 624
RPS_LAST = N - (NS - 1) * RPS


def _sc_aggregate(x, edges_r, zeros):
    mesh = plsc.VectorSubcoreMesh(core_axis_name="c", subcore_axis_name="s")

    @functools.partial(
        pl.kernel,
        out_type=jax.ShapeDtypeStruct((NC, N, D), jnp.float32),
        mesh=mesh,
        compiler_params=pltpu.CompilerParams(use_tc_tiling_on_sc=False),
        scratch_types=[
            pltpu.VMEM((2, NCHUNK, CHUNK), jnp.int32),
            pltpu.VMEM((CHUNK, D), jnp.float32),
            pltpu.VMEM((CHUNK, D), jnp.float32),
            pltpu.VMEM_SHARED((N, D), jnp.float32),
            pltpu.SemaphoreType.DMA,
            pltpu.SemaphoreType.DMA,
        ],
    )
    def k(x_hbm, e_hbm, z_hbm, out_hbm, idx_v, rows0_v, rows1_v, agg_sh,
          sem0, sem1):
        c = lax.axis_index("c")
        s = lax.axis_index("s")
        wid = c * NS + s
        pltpu.sync_copy(e_hbm.at[wid], idx_v)
        r0 = s * RPS

        @pl.when(s < NS - 1)
        def _():
            pltpu.sync_copy(z_hbm.at[pl.ds(0, RPS)], agg_sh.at[pl.ds(r0, RPS)])

        @pl.when(s == NS - 1)
        def _():
            pltpu.sync_copy(
                z_hbm.at[pl.ds(0, RPS_LAST)],
                agg_sh.at[pl.ds((NS - 1) * RPS, RPS_LAST)],
            )

        plsc.subcore_barrier()

        last = NCHUNK - 1

        def gather(j, buf, sem):
            pltpu.async_copy(x_hbm.at[idx_v.at[0, j]], buf, sem)

        def wait(buf, sem):
            pltpu.make_async_copy(x_hbm.at[pl.ds(0, CHUNK)], buf, sem).wait()

        gather(0, rows0_v, sem0)
        gather(1, rows1_v, sem1)

        def body(i, carry):
            j0 = 2 * i
            wait(rows0_v, sem0)
            pltpu.sync_copy(rows0_v, agg_sh.at[idx_v.at[1, j0]], add=True)
            gather(jnp.minimum(j0 + 2, last), rows0_v, sem0)
            wait(rows1_v, sem1)
            pltpu.sync_copy(rows1_v, agg_sh.at[idx_v.at[1, j0 + 1]], add=True)
            gather(jnp.minimum(j0 + 3, last), rows1_v, sem1)
            return carry

        lax.fori_loop(0, NCHUNK // 2, body, 0)
        wait(rows0_v, sem0)
        pltpu.sync_copy(rows0_v, agg_sh.at[idx_v.at[1, last]], add=True)
        wait(rows1_v, sem1)
        plsc.subcore_barrier()

        @pl.when(s < NS - 1)
        def _():
            pltpu.sync_copy(
                agg_sh.at[pl.ds(r0, RPS)], out_hbm.at[c, pl.ds(r0, RPS)]
            )

        @pl.when(s == NS - 1)
        def _():
            pltpu.sync_copy(
                agg_sh.at[pl.ds((NS - 1) * RPS, RPS_LAST)],
                out_hbm.at[c, pl.ds((NS - 1) * RPS, RPS_LAST)],
            )

    return k(x, edges_r, zeros)


def _tc_finish(x, parts, w2d, W1, b1, W2, b2):
    def body(x_ref, p_ref, w_ref, w1_ref, b1_ref, w2_ref, b2_ref, out_ref):
        h = x_ref[...] + p_ref[0] + p_ref[1]
        t = jnp.dot(h, w1_ref[...].T, preferred_element_type=jnp.float32)
        t = jnp.maximum(t + b1_ref[...], 0.0)
        wv = w_ref[...]
        v = jnp.sum(t * wv, axis=0, keepdims=True)
        sw = jnp.sum(wv)
        out = jnp.dot(v, w2_ref[...].T, preferred_element_type=jnp.float32)
        out_ref[...] = out + sw * b2_ref[...]

    return pl.pallas_call(
        body,
        out_shape=jax.ShapeDtypeStruct((1, D), jnp.float32),
    )(x, parts, w2d, W1, b1, W2, b2)


def kernel(x, edge_index, weights, W1, b1, W2, b2):
    edges_r = edge_index.reshape(2, NW, NCHUNK, CHUNK).transpose(1, 0, 2, 3)
    zeros = jnp.zeros((RPS_LAST, D), jnp.float32)
    parts = _sc_aggregate(x, edges_r, zeros)
    out = _tc_finish(x, parts, weights.reshape(N, 1), W1, b1, W2, b2)
    return out.reshape(1, 1, D)


# speedup vs baseline: 4.9389x; 1.0486x over previous
import functools

import jax
import jax.numpy as jnp
from jax import lax
from jax.experimental import pallas as pl
from jax.experimental.pallas import tpu as pltpu
from jax.experimental.pallas import tpu_sc as plsc

N = 10000
E = 320000
D = 128
NC, NS = 2, 16
NW = NC * NS
EPW = E // NW
CHUNK = 100
NCHUNK = EPW // CHUNK
RPS = 624
RPS_LAST = N - (NS - 1) * RPS


def _sc_aggregate(x, edges_r, zeros):
    mesh = plsc.VectorSubcoreMesh(core_axis_name="c", subcore_axis_name="s")

    @functools.partial(
        pl.kernel,
        out_type=jax.ShapeDtypeStruct((NC, N, D), jnp.float32),
        mesh=mesh,
        compiler_params=pltpu.CompilerParams(use_tc_tiling_on_sc=False),
        scratch_types=[
            pltpu.VMEM((2, NCHUNK, CHUNK), jnp.int32),
            pltpu.VMEM((CHUNK, D), jnp.float32),
            pltpu.VMEM((CHUNK, D), jnp.float32),
            pltpu.VMEM_SHARED((N, D), jnp.float32),
            pltpu.SemaphoreType.DMA,
            pltpu.SemaphoreType.DMA,
        ],
    )
    def k(x_hbm, e_hbm, z_hbm, out_hbm, idx_v, rows0_v, rows1_v, agg_sh,
          sem0, sem1):
        c = lax.axis_index("c")
        s = lax.axis_index("s")
        wid = c * NS + s
        pltpu.sync_copy(e_hbm.at[0, wid], idx_v.at[0])
        pltpu.sync_copy(e_hbm.at[1, wid], idx_v.at[1])
        r0 = s * RPS

        @pl.when(s < NS - 1)
        def _():
            pltpu.sync_copy(z_hbm.at[pl.ds(0, RPS)], agg_sh.at[pl.ds(r0, RPS)])

        @pl.when(s == NS - 1)
        def _():
            pltpu.sync_copy(
                z_hbm.at[pl.ds(0, RPS_LAST)],
                agg_sh.at[pl.ds((NS - 1) * RPS, RPS_LAST)],
            )

        plsc.subcore_barrier()

        last = NCHUNK - 1

        def gather(j, buf, sem):
            pltpu.async_copy(x_hbm.at[idx_v.at[0, j]], buf, sem)

        def wait(buf, sem):
            pltpu.make_async_copy(x_hbm.at[pl.ds(0, CHUNK)], buf, sem).wait()

        gather(0, rows0_v, sem0)
        gather(1, rows1_v, sem1)

        def body(i, carry):
            j0 = 2 * i
            wait(rows0_v, sem0)
            pltpu.sync_copy(rows0_v, agg_sh.at[idx_v.at[1, j0]], add=True)
            gather(jnp.minimum(j0 + 2, last), rows0_v, sem0)
            wait(rows1_v, sem1)
            pltpu.sync_copy(rows1_v, agg_sh.at[idx_v.at[1, j0 + 1]], add=True)
            gather(jnp.minimum(j0 + 3, last), rows1_v, sem1)
            return carry

        lax.fori_loop(0, NCHUNK // 2, body, 0)
        wait(rows0_v, sem0)
        wait(rows1_v, sem1)
        plsc.subcore_barrier()

        @pl.when(s < NS - 1)
        def _():
            pltpu.sync_copy(
                agg_sh.at[pl.ds(r0, RPS)], out_hbm.at[c, pl.ds(r0, RPS)]
            )

        @pl.when(s == NS - 1)
        def _():
            pltpu.sync_copy(
                agg_sh.at[pl.ds((NS - 1) * RPS, RPS_LAST)],
                out_hbm.at[c, pl.ds((NS - 1) * RPS, RPS_LAST)],
            )

    return k(x, edges_r, zeros)


def _tc_finish(x, parts, w2d, W1, b1, W2, b2):
    def body(x_ref, p_ref, w_ref, w1_ref, b1_ref, w2_ref, b2_ref, out_ref):
        h = x_ref[...] + p_ref[0] + p_ref[1]
        t = jnp.dot(h, w1_ref[...].T, preferred_element_type=jnp.float32)
        t = jnp.maximum(t + b1_ref[...], 0.0)
        wv = w_ref[...]
        v = jnp.sum(t * wv, axis=0, keepdims=True)
        sw = jnp.sum(wv)
        out = jnp.dot(v, w2_ref[...].T, preferred_element_type=jnp.float32)
        out_ref[...] = out + sw * b2_ref[...]

    return pl.pallas_call(
        body,
        out_shape=jax.ShapeDtypeStruct((1, D), jnp.float32),
    )(x, parts, w2d, W1, b1, W2, b2)


def kernel(x, edge_index, weights, W1, b1, W2, b2):
    edges_r = edge_index.reshape(2, NW, NCHUNK, CHUNK)
    zeros = jnp.zeros((RPS_LAST, D), jnp.float32)
    parts = _sc_aggregate(x, edges_r, zeros)
    out = _tc_finish(x, parts, weights.reshape(N, 1), W1, b1, W2, b2)
    return out.reshape(1, 1, D)
